# baseline (speedup 1.0000x reference)
import functools

import jax
import jax.numpy as jnp
from jax.experimental import pallas as pl
from jax.experimental.pallas import tpu as pltpu

B = 8192
D = 2048
H = 64
E = 8
A = 16

BLK = 512


def _body(inp_ref, act_ref, w1_ref, b1_ref, w2_ref, b2_ref,
          wc_ref, bc_ref, wa_ref, ba_ref, val_ref, lp_ref):
    x = jnp.tanh(jnp.dot(inp_ref[...].astype(jnp.bfloat16),
                         w1_ref[...].astype(jnp.bfloat16),
                         preferred_element_type=jnp.float32) + b1_ref[...])
    x = jnp.tanh(jnp.dot(x.astype(jnp.bfloat16),
                         w2_ref[...].astype(jnp.bfloat16),
                         preferred_element_type=jnp.float32) + b2_ref[...])
    a = act_ref[...]
    onehot = (jax.lax.broadcasted_iota(jnp.int32, (BLK, E), 1) == a
              ).astype(jnp.float32)
    emask = (jax.lax.broadcasted_iota(jnp.int32, (BLK, E * H), 1) // H == a
             ).astype(jnp.float32)
    xb = jnp.concatenate([x] * E, axis=1) * emask
    val_ref[...] = (jnp.dot(xb, wc_ref[...], preferred_element_type=jnp.float32)
                    + jnp.dot(onehot, bc_ref[...],
                              preferred_element_type=jnp.float32))
    logits = (jnp.dot(xb, wa_ref[...], preferred_element_type=jnp.float32)
              + jnp.dot(onehot, ba_ref[...],
                        preferred_element_type=jnp.float32))
    m = jnp.max(logits, axis=1, keepdims=True)
    s = logits - m
    lp_ref[...] = s - jnp.log(jnp.sum(jnp.exp(s), axis=1, keepdims=True))


@functools.partial(jax.jit, static_argnames=())
def kernel(inputs, states, masks, input_action, W1, b1, W2, b2, Wc, bc, Wa, ba):
    act2d = input_action.reshape(B, 1).astype(jnp.int32)
    wc_big = Wc.reshape(E * H, 1)
    wa_big = Wa.reshape(E * H, A)
    grid = (B // BLK,)
    value, log_probs = pl.pallas_call(
        _body,
        grid=grid,
        in_specs=[
            pl.BlockSpec((BLK, D), lambda i: (i, 0)),
            pl.BlockSpec((BLK, 1), lambda i: (i, 0)),
            pl.BlockSpec((D, H), lambda i: (0, 0)),
            pl.BlockSpec((1, H), lambda i: (0, 0)),
            pl.BlockSpec((H, H), lambda i: (0, 0)),
            pl.BlockSpec((1, H), lambda i: (0, 0)),
            pl.BlockSpec((E * H, 1), lambda i: (0, 0)),
            pl.BlockSpec((E, 1), lambda i: (0, 0)),
            pl.BlockSpec((E * H, A), lambda i: (0, 0)),
            pl.BlockSpec((E, A), lambda i: (0, 0)),
        ],
        out_specs=[
            pl.BlockSpec((BLK, 1), lambda i: (i, 0)),
            pl.BlockSpec((BLK, A), lambda i: (i, 0)),
        ],
        out_shape=[
            jax.ShapeDtypeStruct((B, 1), jnp.float32),
            jax.ShapeDtypeStruct((B, A), jnp.float32),
        ],
        compiler_params=pltpu.CompilerParams(
            dimension_semantics=("arbitrary",)),
    )(inputs, act2d, W1, b1.reshape(1, H), W2, b2.reshape(1, H),
      wc_big, bc, wa_big, ba)
    return value, log_probs, states


# speedup vs baseline: 1.0620x; 1.0620x over previous
import functools

import jax
import jax.numpy as jnp
from jax.experimental import pallas as pl
from jax.experimental.pallas import tpu as pltpu

B = 8192
D = 2048
H = 64
E = 8
A = 16

BLK = 512


NSPLIT = 4
DSUB = D // NSPLIT


def _body(inp0_ref, inp1_ref, inp2_ref, inp3_ref, act_ref, w1_ref, b1_ref,
          w2_ref, b2_ref, wc_ref, bc_ref, wa_ref, ba_ref, val_ref, lp_ref):
    w1 = w1_ref[...].astype(jnp.bfloat16)
    acc = jnp.dot(inp0_ref[...].astype(jnp.bfloat16), w1[0 * DSUB:1 * DSUB],
                  preferred_element_type=jnp.float32)
    acc += jnp.dot(inp1_ref[...].astype(jnp.bfloat16), w1[1 * DSUB:2 * DSUB],
                   preferred_element_type=jnp.float32)
    acc += jnp.dot(inp2_ref[...].astype(jnp.bfloat16), w1[2 * DSUB:3 * DSUB],
                   preferred_element_type=jnp.float32)
    acc += jnp.dot(inp3_ref[...].astype(jnp.bfloat16), w1[3 * DSUB:4 * DSUB],
                   preferred_element_type=jnp.float32)
    x = jnp.tanh(acc + b1_ref[...])
    x = jnp.tanh(jnp.dot(x.astype(jnp.bfloat16),
                         w2_ref[...].astype(jnp.bfloat16),
                         preferred_element_type=jnp.float32) + b2_ref[...])
    a = act_ref[...]
    onehot = (jax.lax.broadcasted_iota(jnp.int32, (BLK, E), 1) == a
              ).astype(jnp.float32)
    emask = (jax.lax.broadcasted_iota(jnp.int32, (BLK, E * H), 1) // H == a
             ).astype(jnp.float32)
    xb = jnp.concatenate([x] * E, axis=1) * emask
    val_ref[...] = (jnp.dot(xb, wc_ref[...], preferred_element_type=jnp.float32)
                    + jnp.dot(onehot, bc_ref[...],
                              preferred_element_type=jnp.float32))
    logits = (jnp.dot(xb, wa_ref[...], preferred_element_type=jnp.float32)
              + jnp.dot(onehot, ba_ref[...],
                        preferred_element_type=jnp.float32))
    m = jnp.max(logits, axis=1, keepdims=True)
    s = logits - m
    lp_ref[...] = s - jnp.log(jnp.sum(jnp.exp(s), axis=1, keepdims=True))


@functools.partial(jax.jit, static_argnames=())
def kernel(inputs, states, masks, input_action, W1, b1, W2, b2, Wc, bc, Wa, ba):
    act2d = input_action.reshape(B, 1).astype(jnp.int32)
    wc_big = Wc.reshape(E * H, 1)
    wa_big = Wa.reshape(E * H, A)
    grid = (B // BLK,)
    value, log_probs = pl.pallas_call(
        _body,
        grid=grid,
        in_specs=[
            pl.BlockSpec((BLK, DSUB), lambda i: (i, 0)),
            pl.BlockSpec((BLK, DSUB), lambda i: (i, 1)),
            pl.BlockSpec((BLK, DSUB), lambda i: (i, 2)),
            pl.BlockSpec((BLK, DSUB), lambda i: (i, 3)),
            pl.BlockSpec((BLK, 1), lambda i: (i, 0)),
            pl.BlockSpec((D, H), lambda i: (0, 0)),
            pl.BlockSpec((1, H), lambda i: (0, 0)),
            pl.BlockSpec((H, H), lambda i: (0, 0)),
            pl.BlockSpec((1, H), lambda i: (0, 0)),
            pl.BlockSpec((E * H, 1), lambda i: (0, 0)),
            pl.BlockSpec((E, 1), lambda i: (0, 0)),
            pl.BlockSpec((E * H, A), lambda i: (0, 0)),
            pl.BlockSpec((E, A), lambda i: (0, 0)),
        ],
        out_specs=[
            pl.BlockSpec((BLK, 1), lambda i: (i, 0)),
            pl.BlockSpec((BLK, A), lambda i: (i, 0)),
        ],
        out_shape=[
            jax.ShapeDtypeStruct((B, 1), jnp.float32),
            jax.ShapeDtypeStruct((B, A), jnp.float32),
        ],
        compiler_params=pltpu.CompilerParams(
            dimension_semantics=("arbitrary",)),
    )(inputs, inputs, inputs, inputs, act2d, W1, b1.reshape(1, H), W2,
      b2.reshape(1, H), wc_big, bc, wa_big, ba)
    return value, log_probs, states


# speedup vs baseline: 1.1965x; 1.1267x over previous
import functools

import jax
import jax.numpy as jnp
from jax.experimental import pallas as pl
from jax.experimental.pallas import tpu as pltpu

B = 8192
D = 2048
H = 64
E = 8
A = 16

BLK = 1024


NSPLIT = 4
DSUB = D // NSPLIT


def _body(inp0_ref, inp1_ref, inp2_ref, inp3_ref, act_ref, w1_ref, b1_ref,
          w2_ref, b2_ref, wc_ref, bc_ref, wa_ref, ba_ref, val_ref, lp_ref):
    w1 = w1_ref[...].astype(jnp.bfloat16)
    acc = jnp.dot(inp0_ref[...].astype(jnp.bfloat16), w1[0 * DSUB:1 * DSUB],
                  preferred_element_type=jnp.float32)
    acc += jnp.dot(inp1_ref[...].astype(jnp.bfloat16), w1[1 * DSUB:2 * DSUB],
                   preferred_element_type=jnp.float32)
    acc += jnp.dot(inp2_ref[...].astype(jnp.bfloat16), w1[2 * DSUB:3 * DSUB],
                   preferred_element_type=jnp.float32)
    acc += jnp.dot(inp3_ref[...].astype(jnp.bfloat16), w1[3 * DSUB:4 * DSUB],
                   preferred_element_type=jnp.float32)
    x = jnp.tanh(acc + b1_ref[...])
    x = jnp.tanh(jnp.dot(x.astype(jnp.bfloat16),
                         w2_ref[...].astype(jnp.bfloat16),
                         preferred_element_type=jnp.float32) + b2_ref[...])
    a = act_ref[...]
    onehot = (jax.lax.broadcasted_iota(jnp.int32, (BLK, E), 1) == a
              ).astype(jnp.float32)
    emask = (jax.lax.broadcasted_iota(jnp.int32, (BLK, E * H), 1) // H == a
             ).astype(jnp.float32)
    xb = jnp.concatenate([x] * E, axis=1) * emask
    val_ref[...] = (jnp.dot(xb, wc_ref[...], preferred_element_type=jnp.float32)
                    + jnp.dot(onehot, bc_ref[...],
                              preferred_element_type=jnp.float32))
    logits = (jnp.dot(xb, wa_ref[...], preferred_element_type=jnp.float32)
              + jnp.dot(onehot, ba_ref[...],
                        preferred_element_type=jnp.float32))
    m = jnp.max(logits, axis=1, keepdims=True)
    s = logits - m
    lp_ref[...] = s - jnp.log(jnp.sum(jnp.exp(s), axis=1, keepdims=True))


@functools.partial(jax.jit, static_argnames=())
def kernel(inputs, states, masks, input_action, W1, b1, W2, b2, Wc, bc, Wa, ba):
    act2d = input_action.reshape(B, 1).astype(jnp.int32)
    wc_big = Wc.reshape(E * H, 1)
    wa_big = Wa.reshape(E * H, A)
    grid = (B // BLK,)
    value, log_probs = pl.pallas_call(
        _body,
        grid=grid,
        in_specs=[
            pl.BlockSpec((BLK, DSUB), lambda i: (i, 0)),
            pl.BlockSpec((BLK, DSUB), lambda i: (i, 1)),
            pl.BlockSpec((BLK, DSUB), lambda i: (i, 2)),
            pl.BlockSpec((BLK, DSUB), lambda i: (i, 3)),
            pl.BlockSpec((BLK, 1), lambda i: (i, 0)),
            pl.BlockSpec((D, H), lambda i: (0, 0)),
            pl.BlockSpec((1, H), lambda i: (0, 0)),
            pl.BlockSpec((H, H), lambda i: (0, 0)),
            pl.BlockSpec((1, H), lambda i: (0, 0)),
            pl.BlockSpec((E * H, 1), lambda i: (0, 0)),
            pl.BlockSpec((E, 1), lambda i: (0, 0)),
            pl.BlockSpec((E * H, A), lambda i: (0, 0)),
            pl.BlockSpec((E, A), lambda i: (0, 0)),
        ],
        out_specs=[
            pl.BlockSpec((BLK, 1), lambda i: (i, 0)),
            pl.BlockSpec((BLK, A), lambda i: (i, 0)),
        ],
        out_shape=[
            jax.ShapeDtypeStruct((B, 1), jnp.float32),
            jax.ShapeDtypeStruct((B, A), jnp.float32),
        ],
        compiler_params=pltpu.CompilerParams(
            dimension_semantics=("arbitrary",)),
    )(inputs, inputs, inputs, inputs, act2d, W1, b1.reshape(1, H), W2,
      b2.reshape(1, H), wc_big, bc, wa_big, ba)
    return value, log_probs, states
